# baseline (speedup 1.0000x reference)
import functools

import jax
import jax.numpy as jnp
from jax import lax
from jax.experimental import pallas as pl
from jax.experimental.pallas import tpu as pltpu
from jax.experimental.pallas import tpu_sc as plsc

S, D, H, E = 2048, 768, 768, 8
BLK = 256
NBLK = -(-(S + E * (BLK - 1)) // BLK)
NPAD = NBLK * BLK
NC, NS = 2, 16
NW = NC * NS
TPB = S // NW
NCH = 4
CH = TPB // NCH


def _router_body(x_ref, rw_ref, rb_ref, pos_ref, be_ref, w_ref, aux_ref):
    x = x_ref[...]
    rw = rw_ref[...]
    rb = rb_ref[...]
    logits = lax.dot_general(x, rw, (((1,), (1,)), ((), ())),
                             preferred_element_type=jnp.float32)
    logits = logits + rb
    aux_ref[...] = (0.01 / (S * E)) * jnp.sum(
        logits * logits, axis=(0, 1), keepdims=True)

    scores = jax.nn.sigmoid(logits)
    m = jnp.max(scores, axis=1, keepdims=True)
    eidx = lax.broadcasted_iota(jnp.int32, (S, E), 1)
    choice = jnp.min(jnp.where(scores == m, eidx, E), axis=1, keepdims=True)
    weight = m / (m + 1e-6)
    w_ref[...] = jnp.broadcast_to(weight, (S, 128))

    onehot = (eidx == choice).astype(jnp.float32)
    c = onehot
    sh = 1
    while sh < S:
        c = c + jnp.concatenate([jnp.zeros((sh, E), jnp.float32), c[:-sh]], 0)
        sh *= 2
    rank = c - onehot
    counts = c[S - 1:S, :]
    pc = jnp.floor((counts + (BLK - 1)) * (1.0 / BLK)) * BLK
    c2 = pc
    sh = 1
    while sh < E:
        c2 = c2 + jnp.concatenate(
            [jnp.zeros((1, sh), jnp.float32), c2[:, :-sh]], 1)
        sh *= 2
    ps = c2 - pc
    posf = jnp.sum(onehot * (rank + ps), axis=1, keepdims=True)
    pos_ref[...] = posf.astype(jnp.int32)

    pend = ps + pc
    jb = lax.broadcasted_iota(jnp.int32, (NBLK, 1), 0).astype(jnp.float32) * BLK
    bef = jnp.sum((jb >= pend).astype(jnp.float32), axis=1, keepdims=True)
    be_ref[...] = jnp.minimum(bef, E - 1).astype(jnp.int32)


_router_call = pl.pallas_call(
    _router_body,
    out_shape=(
        jax.ShapeDtypeStruct((S, 1), jnp.int32),
        jax.ShapeDtypeStruct((NBLK, 1), jnp.int32),
        jax.ShapeDtypeStruct((S, 128), jnp.float32),
        jax.ShapeDtypeStruct((1, 1), jnp.float32),
    ),
)


@functools.cache
def _get_dispatch_call():
    mesh = plsc.VectorSubcoreMesh(core_axis_name="c", subcore_axis_name="s")

    @functools.partial(
        pl.kernel,
        mesh=mesh,
        out_type=[
            jax.ShapeDtypeStruct((NPAD, D), jnp.float32),
            jax.ShapeDtypeStruct((NPAD, 128), jnp.float32),
        ],
        scratch_types=[
            pltpu.VMEM((CH,), jnp.int32),
            pltpu.VMEM((CH,), jnp.int32),
            pltpu.VMEM((CH,), jnp.int32),
            pltpu.VMEM((CH,), jnp.int32),
            pltpu.VMEM((TPB, D), jnp.float32),
            pltpu.VMEM((TPB, 128), jnp.float32),
            pltpu.SemaphoreType.DMA,
            pltpu.SemaphoreType.DMA,
            pltpu.SemaphoreType.DMA,
            pltpu.SemaphoreType.DMA,
            pltpu.SemaphoreType.DMA,
            pltpu.SemaphoreType.DMA,
        ],
    )
    def _dispatch_call(x_hbm, pos_hbm, w_hbm, xs_hbm, ws_hbm,
                       i0, i1, i2, i3, rows_v, wrows_v,
                       s0, s1, s2, s3, sw, si):
        wid = lax.axis_index("s") * NC + lax.axis_index("c")
        base = wid * TPB
        idx = (i0, i1, i2, i3)
        sx = (s0, s1, s2, s3)
        cw = pltpu.async_copy(w_hbm.at[pl.ds(base, TPB)], wrows_v, sw)
        cps = [pltpu.async_copy(pos_hbm.at[pl.ds(base + c * CH, CH)],
                                idx[c], si) for c in range(NCH)]
        cxs = [pltpu.async_copy(x_hbm.at[pl.ds(base + c * CH, CH)],
                                rows_v.at[pl.ds(c * CH, CH)], sx[c])
               for c in range(NCH)]
        for c in range(NCH):
            cps[c].wait()
        scs = []
        for c in range(NCH):
            cxs[c].wait()
            scs.append(pltpu.async_copy(rows_v.at[pl.ds(c * CH, CH)],
                                        xs_hbm.at[idx[c]], sx[c]))
        cw.wait()
        wcs = []
        for c in range(NCH):
            wcs.append(pltpu.async_copy(wrows_v.at[pl.ds(c * CH, CH)],
                                        ws_hbm.at[idx[c]], sw))
        for c in range(NCH):
            scs[c].wait()
            wcs[c].wait()

    return _dispatch_call


def _expert_body(be_ref, xs_ref, w12_hbm, w3_hbm, ws_ref, o_ref,
                 w12_buf, w3_buf, s12, s3, st_ref):
    j = pl.program_id(0)
    e = be_ref[j]

    @pl.when(j == 0)
    def _():
        st_ref[0] = jnp.int32(-1)
        st_ref[9] = jnp.int32(0)

    for d in (0, 1, 2, 3):
        t = be_ref[jnp.minimum(j + d, NBLK - 1)]

        @pl.when(t > st_ref[0])
        def _():
            k = st_ref[9]
            slot = lax.rem(k, 4)
            pltpu.make_async_copy(w12_hbm.at[t], w12_buf.at[slot],
                                  s12.at[slot]).start()
            pltpu.make_async_copy(w3_hbm.at[t], w3_buf.at[slot],
                                  s3.at[slot]).start()
            st_ref[1 + t] = slot
            st_ref[9] = k + 1
            st_ref[0] = t

    slot = st_ref[1 + e]
    first = jnp.logical_or(j == 0, be_ref[jnp.maximum(j - 1, 0)] != e)

    @pl.when(first)
    def _():
        pltpu.make_async_copy(w12_hbm.at[e], w12_buf.at[slot],
                              s12.at[slot]).wait()
        pltpu.make_async_copy(w3_hbm.at[e], w3_buf.at[slot],
                              s3.at[slot]).wait()

    xb = xs_ref[...]
    w12b = w12_buf[slot]
    h12 = lax.dot_general(xb, w12b, (((1,), (1,)), ((), ())),
                          preferred_element_type=jnp.float32)
    x1 = h12[:, :H]
    x2 = h12[:, H:]
    hidden = x1 * jax.nn.sigmoid(x1) * x2
    w3b = w3_buf[slot]
    ob = lax.dot_general(hidden, w3b, (((1,), (1,)), ((), ())),
                         preferred_element_type=jnp.float32)
    o_ref[...] = ob * ws_ref[:, 0:1]


_expert_call = pl.pallas_call(
    _expert_body,
    grid_spec=pltpu.PrefetchScalarGridSpec(
        num_scalar_prefetch=1,
        grid=(NBLK,),
        in_specs=[
            pl.BlockSpec((BLK, D), lambda i, be: (i, 0)),
            pl.BlockSpec(memory_space=pl.ANY),
            pl.BlockSpec(memory_space=pl.ANY),
            pl.BlockSpec((BLK, 128), lambda i, be: (i, 0)),
        ],
        out_specs=pl.BlockSpec((BLK, D), lambda i, be: (i, 0)),
        scratch_shapes=[
            pltpu.VMEM((4, 2 * H, D), jnp.float32),
            pltpu.VMEM((4, D, H), jnp.float32),
            pltpu.SemaphoreType.DMA((4,)),
            pltpu.SemaphoreType.DMA((4,)),
            pltpu.SMEM((10,), jnp.int32),
        ],
    ),
    out_shape=jax.ShapeDtypeStruct((NPAD, D), jnp.float32),
)


@functools.cache
def _get_combine_call():
    mesh = plsc.VectorSubcoreMesh(core_axis_name="c", subcore_axis_name="s")

    @functools.partial(
        pl.kernel,
        mesh=mesh,
        out_type=jax.ShapeDtypeStruct((S, D), jnp.float32),
        scratch_types=[
            pltpu.VMEM((CH,), jnp.int32),
            pltpu.VMEM((CH,), jnp.int32),
            pltpu.VMEM((CH,), jnp.int32),
            pltpu.VMEM((CH,), jnp.int32),
            pltpu.VMEM((TPB, D), jnp.float32),
            pltpu.SemaphoreType.DMA,
            pltpu.SemaphoreType.DMA,
            pltpu.SemaphoreType.DMA,
            pltpu.SemaphoreType.DMA,
            pltpu.SemaphoreType.DMA,
        ],
    )
    def _combine_call(osc_hbm, pos_hbm, out_hbm,
                      i0, i1, i2, i3, rows_v, s0, s1, s2, s3, si):
        wid = lax.axis_index("s") * NC + lax.axis_index("c")
        base = wid * TPB
        idx = (i0, i1, i2, i3)
        sx = (s0, s1, s2, s3)
        cps = [pltpu.async_copy(pos_hbm.at[pl.ds(base + c * CH, CH)],
                                idx[c], si) for c in range(NCH)]
        for c in range(NCH):
            cps[c].wait()
        gcs = [pltpu.async_copy(osc_hbm.at[idx[c]],
                                rows_v.at[pl.ds(c * CH, CH)], sx[c])
               for c in range(NCH)]
        ocs = []
        for c in range(NCH):
            gcs[c].wait()
            ocs.append(pltpu.async_copy(rows_v.at[pl.ds(c * CH, CH)],
                                        out_hbm.at[pl.ds(base + c * CH, CH)],
                                        sx[c]))
        for c in range(NCH):
            ocs[c].wait()

    return _combine_call


def kernel(x, router_w, router_b, w12, w3):
    x2 = x.reshape(S, D)
    rb2 = router_b.reshape(1, E)
    pos2, be2, wrows, aux = _router_call(x2, router_w, rb2)
    pos = pos2.reshape(S)
    be = be2.reshape(NBLK)
    xs, ws = _get_dispatch_call()(x2, pos, wrows)
    osc = _expert_call(be, xs, w12, w3, ws)
    out = _get_combine_call()(osc, pos)
    return out.reshape(1, S, D), aux.reshape(())


# speedup vs baseline: 1.0004x; 1.0004x over previous
import functools

import jax
import jax.numpy as jnp
from jax import lax
from jax.experimental import pallas as pl
from jax.experimental.pallas import tpu as pltpu
from jax.experimental.pallas import tpu_sc as plsc

S, D, H, E = 2048, 768, 768, 8
BLK = 256
NBLK = -(-(S + E * (BLK - 1)) // BLK)
NPAD = NBLK * BLK
NC, NS = 2, 16
NW = NC * NS
TPB = S // NW


def _router_body(x_ref, rw_ref, rb_ref, pos_ref, be_ref, w_ref, aux_ref):
    x = x_ref[...]
    rw = rw_ref[...]
    rb = rb_ref[...]
    logits = lax.dot_general(x, rw, (((1,), (1,)), ((), ())),
                             preferred_element_type=jnp.float32)
    logits = logits + rb
    aux_ref[...] = (0.01 / (S * E)) * jnp.sum(
        logits * logits, axis=(0, 1), keepdims=True)

    scores = jax.nn.sigmoid(logits)
    m = jnp.max(scores, axis=1, keepdims=True)
    eidx = lax.broadcasted_iota(jnp.int32, (S, E), 1)
    choice = jnp.min(jnp.where(scores == m, eidx, E), axis=1, keepdims=True)
    weight = m / (m + 1e-6)
    w_ref[...] = jnp.broadcast_to(weight, (S, 128))

    onehot = (eidx == choice).astype(jnp.float32)
    c = onehot
    sh = 1
    while sh < S:
        c = c + jnp.concatenate([jnp.zeros((sh, E), jnp.float32), c[:-sh]], 0)
        sh *= 2
    rank = c - onehot
    counts = c[S - 1:S, :]
    pc = jnp.floor((counts + (BLK - 1)) * (1.0 / BLK)) * BLK
    c2 = pc
    sh = 1
    while sh < E:
        c2 = c2 + jnp.concatenate(
            [jnp.zeros((1, sh), jnp.float32), c2[:, :-sh]], 1)
        sh *= 2
    ps = c2 - pc
    posf = jnp.sum(onehot * (rank + ps), axis=1, keepdims=True)
    pos_ref[...] = posf.astype(jnp.int32)

    pend = ps + pc
    jb = lax.broadcasted_iota(jnp.int32, (NBLK, 1), 0).astype(jnp.float32) * BLK
    bef = jnp.sum((jb >= pend).astype(jnp.float32), axis=1, keepdims=True)
    be_ref[...] = jnp.minimum(bef, E - 1).astype(jnp.int32)


_router_call = pl.pallas_call(
    _router_body,
    out_shape=(
        jax.ShapeDtypeStruct((S, 1), jnp.int32),
        jax.ShapeDtypeStruct((NBLK, 1), jnp.int32),
        jax.ShapeDtypeStruct((S, 128), jnp.float32),
        jax.ShapeDtypeStruct((1, 1), jnp.float32),
    ),
)


@functools.cache
def _get_dispatch_call():
    mesh = plsc.VectorSubcoreMesh(core_axis_name="c", subcore_axis_name="s")

    @functools.partial(
        pl.kernel,
        mesh=mesh,
        out_type=[
            jax.ShapeDtypeStruct((NPAD, D), jnp.float32),
            jax.ShapeDtypeStruct((NPAD, 128), jnp.float32),
        ],
        scratch_types=[
            pltpu.VMEM((TPB,), jnp.int32),
            pltpu.VMEM((TPB, D), jnp.float32),
            pltpu.VMEM((TPB, 128), jnp.float32),
            pltpu.SemaphoreType.DMA,
            pltpu.SemaphoreType.DMA,
            pltpu.SemaphoreType.DMA,
        ],
    )
    def _dispatch_call(x_hbm, pos_hbm, w_hbm, xs_hbm, ws_hbm,
                       idx_v, rows_v, wrows_v, sem1, sem2, sem3):
        wid = lax.axis_index("s") * NC + lax.axis_index("c")
        base = wid * TPB
        cp = pltpu.async_copy(pos_hbm.at[pl.ds(base, TPB)], idx_v, sem3)
        cx = pltpu.async_copy(x_hbm.at[pl.ds(base, TPB)], rows_v, sem1)
        cw = pltpu.async_copy(w_hbm.at[pl.ds(base, TPB)], wrows_v, sem2)
        cp.wait()
        cx.wait()
        c1 = pltpu.async_copy(rows_v, xs_hbm.at[idx_v], sem1)
        cw.wait()
        c2 = pltpu.async_copy(wrows_v, ws_hbm.at[idx_v], sem2)
        c1.wait()
        c2.wait()

    return _dispatch_call


def _expert_body(be_ref, xs_ref, w12_hbm, w3_hbm, ws_ref, o_ref,
                 w12_buf, w3_buf, s12, s3, st_ref):
    j = pl.program_id(0)
    e = be_ref[j]

    @pl.when(j == 0)
    def _():
        st_ref[0] = jnp.int32(-1)
        st_ref[9] = jnp.int32(0)

    for d in (0, 1, 2, 3):
        t = be_ref[jnp.minimum(j + d, NBLK - 1)]

        @pl.when(t > st_ref[0])
        def _():
            k = st_ref[9]
            slot = lax.rem(k, 4)
            pltpu.make_async_copy(w12_hbm.at[t], w12_buf.at[slot],
                                  s12.at[slot]).start()
            pltpu.make_async_copy(w3_hbm.at[t], w3_buf.at[slot],
                                  s3.at[slot]).start()
            st_ref[1 + t] = slot
            st_ref[9] = k + 1
            st_ref[0] = t

    slot = st_ref[1 + e]
    first = jnp.logical_or(j == 0, be_ref[jnp.maximum(j - 1, 0)] != e)

    @pl.when(first)
    def _():
        pltpu.make_async_copy(w12_hbm.at[e], w12_buf.at[slot],
                              s12.at[slot]).wait()
        pltpu.make_async_copy(w3_hbm.at[e], w3_buf.at[slot],
                              s3.at[slot]).wait()

    xb = xs_ref[...]
    w12b = w12_buf[slot]
    h12 = lax.dot_general(xb, w12b, (((1,), (1,)), ((), ())),
                          preferred_element_type=jnp.float32)
    x1 = h12[:, :H]
    x2 = h12[:, H:]
    hidden = x1 * jax.nn.sigmoid(x1) * x2
    w3b = w3_buf[slot]
    ob = lax.dot_general(hidden, w3b, (((1,), (1,)), ((), ())),
                         preferred_element_type=jnp.float32)
    o_ref[...] = ob * ws_ref[:, 0:1]


_expert_call = pl.pallas_call(
    _expert_body,
    grid_spec=pltpu.PrefetchScalarGridSpec(
        num_scalar_prefetch=1,
        grid=(NBLK,),
        in_specs=[
            pl.BlockSpec((BLK, D), lambda i, be: (i, 0)),
            pl.BlockSpec(memory_space=pl.ANY),
            pl.BlockSpec(memory_space=pl.ANY),
            pl.BlockSpec((BLK, 128), lambda i, be: (i, 0)),
        ],
        out_specs=pl.BlockSpec((BLK, D), lambda i, be: (i, 0)),
        scratch_shapes=[
            pltpu.VMEM((4, 2 * H, D), jnp.float32),
            pltpu.VMEM((4, D, H), jnp.float32),
            pltpu.SemaphoreType.DMA((4,)),
            pltpu.SemaphoreType.DMA((4,)),
            pltpu.SMEM((10,), jnp.int32),
        ],
    ),
    out_shape=jax.ShapeDtypeStruct((NPAD, D), jnp.float32),
)


@functools.cache
def _get_combine_call():
    mesh = plsc.VectorSubcoreMesh(core_axis_name="c", subcore_axis_name="s")

    @functools.partial(
        pl.kernel,
        mesh=mesh,
        out_type=jax.ShapeDtypeStruct((S, D), jnp.float32),
        scratch_types=[
            pltpu.VMEM((TPB,), jnp.int32),
            pltpu.VMEM((TPB, D), jnp.float32),
            pltpu.SemaphoreType.DMA,
        ],
    )
    def _combine_call(osc_hbm, pos_hbm, out_hbm, idx_v, rows_v, sem):
        wid = lax.axis_index("s") * NC + lax.axis_index("c")
        base = wid * TPB
        pltpu.sync_copy(pos_hbm.at[pl.ds(base, TPB)], idx_v)
        pltpu.async_copy(osc_hbm.at[idx_v], rows_v, sem).wait()
        pltpu.sync_copy(rows_v, out_hbm.at[pl.ds(base, TPB)])

    return _combine_call


def kernel(x, router_w, router_b, w12, w3):
    x2 = x.reshape(S, D)
    rb2 = router_b.reshape(1, E)
    pos2, be2, wrows, aux = _router_call(x2, router_w, rb2)
    pos = pos2.reshape(S)
    be = be2.reshape(NBLK)
    xs, ws = _get_dispatch_call()(x2, pos, wrows)
    osc = _expert_call(be, xs, w12, w3, ws)
    out = _get_combine_call()(osc, pos)
    return out.reshape(1, S, D), aux.reshape(())
